# Initial kernel scaffold; baseline (speedup 1.0000x reference)
import jax
import jax.numpy as jnp
from jax.experimental import pallas as pl


def kernel(x, edge_index, params):
    raise NotImplementedError("write your pallas kernel here")


# speedup vs baseline: 5.5330x; 5.5330x over previous
import functools

import jax
import jax.numpy as jnp
from jax import lax
from jax.experimental import pallas as pl
from jax.experimental.pallas import tpu as pltpu

HEADS = 8
C = 256
N = 10000
E = 160000
D = C // HEADS

_BN = 1000


def _mm_bias_body(x_ref, w_ref, b_ref, o_ref):
    o_ref[...] = (
        jnp.dot(x_ref[...], w_ref[...], preferred_element_type=jnp.float32)
        + b_ref[...]
    )


def _mm_bias(x, w, b):
    n, c = x.shape
    k = w.shape[1]
    return pl.pallas_call(
        _mm_bias_body,
        grid=(n // _BN,),
        in_specs=[
            pl.BlockSpec((_BN, c), lambda i: (i, 0)),
            pl.BlockSpec((c, k), lambda i: (0, 0)),
            pl.BlockSpec((1, k), lambda i: (0, 0)),
        ],
        out_specs=pl.BlockSpec((_BN, k), lambda i: (i, 0)),
        out_shape=jax.ShapeDtypeStruct((n, k), jnp.float32),
    )(x, w, b.reshape(1, k))


def _ln(h, g, b, eps=1e-5):
    m = jnp.mean(h, axis=-1, keepdims=True)
    v = jnp.mean((h - m) ** 2, axis=-1, keepdims=True)
    return (h - m) * jax.lax.rsqrt(v + eps) * g + b


def _post_body(x_ref, msg_ref, skip_ref, g1_ref, b1_ref, w1_ref, bf1_ref,
               w2_ref, bf2_ref, g2_ref, b2_ref, o_ref):
    h = x_ref[...] + msg_ref[...] + skip_ref[...]
    h = _ln(h, g1_ref[...], b1_ref[...])
    f = jnp.dot(
        jax.nn.relu(
            jnp.dot(h, w1_ref[...], preferred_element_type=jnp.float32)
            + bf1_ref[...]
        ),
        w2_ref[...],
        preferred_element_type=jnp.float32,
    ) + bf2_ref[...]
    o_ref[...] = _ln(h + f, g2_ref[...], b2_ref[...])


def _post_block(x, msg, skip, p):
    c = C
    vec = lambda a: a.reshape(1, -1)
    return pl.pallas_call(
        _post_body,
        grid=(N // _BN,),
        in_specs=[
            pl.BlockSpec((_BN, c), lambda i: (i, 0)),
            pl.BlockSpec((_BN, c), lambda i: (i, 0)),
            pl.BlockSpec((_BN, c), lambda i: (i, 0)),
            pl.BlockSpec((1, c), lambda i: (0, 0)),
            pl.BlockSpec((1, c), lambda i: (0, 0)),
            pl.BlockSpec((c, 4 * c), lambda i: (0, 0)),
            pl.BlockSpec((1, 4 * c), lambda i: (0, 0)),
            pl.BlockSpec((4 * c, c), lambda i: (0, 0)),
            pl.BlockSpec((1, c), lambda i: (0, 0)),
            pl.BlockSpec((1, c), lambda i: (0, 0)),
            pl.BlockSpec((1, c), lambda i: (0, 0)),
        ],
        out_specs=pl.BlockSpec((_BN, c), lambda i: (i, 0)),
        out_shape=jax.ShapeDtypeStruct((N, c), jnp.float32),
    )(x, msg, skip, vec(p['ln1_g']), vec(p['ln1_b']), p['W1'], vec(p['b1']),
      p['W2'], vec(p['bf2']), vec(p['ln2_g']), vec(p['ln2_b']))


def _edge_phase(q, k, v, src, dst):
    alpha = jnp.sum(
        (q[dst] * k[src]).reshape(E, HEADS, D), axis=-1
    ) / jnp.sqrt(jnp.float32(D))
    amax = jax.ops.segment_max(alpha, dst, num_segments=N)
    amax = jnp.where(jnp.isfinite(amax), amax, 0.0)
    ex = jnp.exp(alpha - amax[dst])
    denom = jax.ops.segment_sum(ex, dst, num_segments=N)
    attn = ex / (denom[dst] + 1e-16)
    msg = v[src].reshape(E, HEADS, D) * attn[..., None]
    return jax.ops.segment_sum(msg.reshape(E, C), dst, num_segments=N)


def kernel(x, edge_index, params):
    src = edge_index[0].astype(jnp.int32)
    dst = edge_index[1].astype(jnp.int32)
    x = _mm_bias(x, params['in_W'], params['in_b'])
    for l in range(2):
        p = params['blocks'][l]
        wqkvs = jnp.concatenate([p['Wq'], p['Wk'], p['Wv'], p['Wskip']], axis=1)
        bqkvs = jnp.concatenate([p['bq'], p['bk'], p['bv'], p['bskip']])
        qkvs = _mm_bias(x, wqkvs, bqkvs)
        q, k, v, skip = (qkvs[:, :C], qkvs[:, C:2 * C],
                         qkvs[:, 2 * C:3 * C], qkvs[:, 3 * C:])
        msg = _edge_phase(q, k, v, src, dst)
        pp = dict(p)
        pp['bf2'] = p['b2']
        x = _post_block(x, msg, skip, pp)
    return _mm_bias(x, params['out_W'], params['out_b'])


# speedup vs baseline: 14.3747x; 2.5980x over previous
import functools

import jax
import jax.numpy as jnp
from jax import lax
from jax.experimental import pallas as pl
from jax.experimental.pallas import tpu as pltpu
from jax.experimental.pallas import tpu_sc as plsc

HEADS = 8
C = 256
N = 10000
E = 160000
D = C // HEADS

_TILES = 16
_EPT = E // _TILES
_G = 80
_NCH = _EPT // _G
_HPC = HEADS // 2
_NPAD = 10240
_ROWS_PT = _NPAD // _TILES
_NHALF = _NPAD // 2
_HROWS_PT = _NHALF // _TILES
_HC = C // 2
_DROWS = _NPAD // 8

_BN = 1000


def _mm_bias_body(x_ref, w_ref, b_ref, o_ref):
    o_ref[...] = (
        jnp.dot(x_ref[...], w_ref[...], preferred_element_type=jnp.float32)
        + b_ref[...]
    )


def _mm_bias(x, w, b):
    n, c = x.shape
    k = w.shape[1]
    return pl.pallas_call(
        _mm_bias_body,
        grid=(n // _BN,),
        in_specs=[
            pl.BlockSpec((_BN, c), lambda i: (i, 0)),
            pl.BlockSpec((c, k), lambda i: (0, 0)),
            pl.BlockSpec((1, k), lambda i: (0, 0)),
        ],
        out_specs=pl.BlockSpec((_BN, k), lambda i: (i, 0)),
        out_shape=jax.ShapeDtypeStruct((n, k), jnp.float32),
    )(x, w, b.reshape(1, k))


def _ln(h, g, b, eps=1e-5):
    m = jnp.mean(h, axis=-1, keepdims=True)
    v = jnp.mean((h - m) ** 2, axis=-1, keepdims=True)
    return (h - m) * jax.lax.rsqrt(v + eps) * g + b


def _post_body(x_ref, msg_ref, den_ref, skip_ref, g1_ref, b1_ref, w1_ref,
               bf1_ref, w2_ref, bf2_ref, g2_ref, b2_ref, o_ref):
    rec = 1.0 / (den_ref[...] + 1e-16)
    rec = jnp.broadcast_to(rec[:, :, None], (_BN, HEADS, D)).reshape(_BN, C)
    h = x_ref[...] + msg_ref[...] * rec + skip_ref[...]
    h = _ln(h, g1_ref[...], b1_ref[...])
    f = jnp.dot(
        jax.nn.relu(
            jnp.dot(h, w1_ref[...], preferred_element_type=jnp.float32)
            + bf1_ref[...]
        ),
        w2_ref[...],
        preferred_element_type=jnp.float32,
    ) + bf2_ref[...]
    o_ref[...] = _ln(h + f, g2_ref[...], b2_ref[...])


def _post_block(x, msg, den, skip, p):
    c = C
    vec = lambda a: a.reshape(1, -1)
    return pl.pallas_call(
        _post_body,
        grid=(N // _BN,),
        in_specs=[
            pl.BlockSpec((_BN, c), lambda i: (i, 0)),
            pl.BlockSpec((_BN, c), lambda i: (i, 0)),
            pl.BlockSpec((_BN, HEADS), lambda i: (i, 0)),
            pl.BlockSpec((_BN, c), lambda i: (i, 0)),
            pl.BlockSpec((1, c), lambda i: (0, 0)),
            pl.BlockSpec((1, c), lambda i: (0, 0)),
            pl.BlockSpec((c, 4 * c), lambda i: (0, 0)),
            pl.BlockSpec((1, 4 * c), lambda i: (0, 0)),
            pl.BlockSpec((4 * c, c), lambda i: (0, 0)),
            pl.BlockSpec((1, c), lambda i: (0, 0)),
            pl.BlockSpec((1, c), lambda i: (0, 0)),
            pl.BlockSpec((1, c), lambda i: (0, 0)),
        ],
        out_specs=pl.BlockSpec((_BN, c), lambda i: (i, 0)),
        out_shape=jax.ShapeDtypeStruct((N, c), jnp.float32),
    )(x, msg, den, skip, vec(p['ln1_g']), vec(p['ln1_b']), p['W1'],
      vec(p['b1']), p['W2'], vec(p['bf2']), vec(p['ln2_g']), vec(p['ln2_b']))


def _iota16():
    return lax.iota(jnp.int32, 16)


def _shuf(u, idx):
    return lax.gather(
        u, idx[:, None],
        lax.GatherDimensionNumbers(
            offset_dims=(), collapsed_slice_dims=(0,), start_index_map=(0,)),
        slice_sizes=(1,),
        mode=lax.GatherScatterMode.PROMISE_IN_BOUNDS)


def _bitrev4():
    i = _iota16()
    return ((i & 1) << 3) | ((i & 2) << 1) | ((i & 4) >> 1) | ((i & 8) >> 3)


def _tree_sum16(vs):
    iota = _iota16()
    cur = list(vs)
    d = 8
    while len(cur) > 1:
        idx = iota ^ d
        mask = (iota & d) == 0
        nxt = []
        for j in range(0, len(cur), 2):
            u, w = cur[j], cur[j + 1]
            su = u + _shuf(u, idx)
            sw = w + _shuf(w, idx)
            nxt.append(jnp.where(mask, su, sw))
        cur = nxt
        d //= 2
    return _shuf(cur[0], _bitrev4())


def _edge_body(table, srch, dsth, msg_un, den_out, exr, srcall, dstall,
               qd, ks, vr, exd, alphabuf, qidx, kidx, vidx, didx, didx2,
               acc, den_sh, sem1, sem2, sem3):
    c = lax.axis_index("c")
    s = lax.axis_index("s")
    e0 = s * _EPT
    cN = c * N
    cE4 = c * 4 * E
    lane = _iota16()
    scale = jnp.float32(1.0 / (D ** 0.5))
    zero = jnp.zeros((16,), jnp.float32)

    pltpu.sync_copy(srch.at[pl.ds(e0, _EPT)], srcall)
    pltpu.sync_copy(dsth.at[pl.ds(e0, _EPT)], dstall)

    def zbuf(ref):
        def zi(i, _):
            ref[i // 8, pl.ds((i % 8) * 16, 16)] = zero
            return 0
        lax.fori_loop(0, _G * 8, zi, 0)

    zbuf(exd)
    zbuf(ks)
    for t in range(_HROWS_PT // _G):
        pltpu.sync_copy(ks, acc.at[pl.ds(s * _HROWS_PT + t * _G, _G)])
    pltpu.sync_copy(ks, den_sh.at[pl.ds(s * (_DROWS // _TILES), _G)])

    @pl.when(s == _TILES - 1)
    def _zero_dump():
        pltpu.sync_copy(ks.at[pl.ds(0, 8)], acc.at[pl.ds(_NHALF, 8)])
    plsc.subcore_barrier()

    def copy_idx(src_ref, base, dst_ref, off=None, half=None, shift=None):
        for j in range(_G // 16):
            v = src_ref[pl.ds(base + j * 16, 16)]
            if off is not None:
                v = v + off
            if half is not None:
                rel = v - half * _NHALF
                ok = (rel >= 0) & (rel < _NHALF)
                v = jnp.where(ok, rel, _NHALF)
            if shift is not None:
                v = v >> shift
            dst_ref[pl.ds(j * 16, 16)] = v

    def sweep0(ci, _):
        base = ci * _G
        copy_idx(dstall, base, qidx, off=cN)
        copy_idx(srcall, base, kidx, off=2 * N + cN)
        copy_idx(srcall, base, vidx, off=4 * N + cN)
        copy_idx(dstall, base, didx, shift=3)
        copy_idx(dstall, base, didx2, half=0)
        cp1 = pltpu.async_copy(table.at[qidx], qd, sem1)
        cp2 = pltpu.async_copy(table.at[kidx], ks, sem2)
        cp3 = pltpu.async_copy(table.at[vidx], vr, sem3)
        cp1.wait()
        cp2.wait()

        def grp(g, _):
            dv = dstall[pl.ds(base + g * 16, 16)]
            col0 = (dv & 7) << 4
            for h in range(_HPC):
                vs = []
                for e in range(16):
                    r = g * 16 + e
                    a = qd[r, pl.ds(32 * h, 16)] * ks[r, pl.ds(32 * h, 16)]
                    b = (qd[r, pl.ds(32 * h + 16, 16)]
                         * ks[r, pl.ds(32 * h + 16, 16)])
                    vs.append(a + b)
                exv = jnp.exp(_tree_sum16(vs) * scale)
                alphabuf[h, pl.ds(g * 16, 16)] = exv
                plsc.store_scatter(exd, [g * 16 + lane, col0 + h], exv)
            return 0
        lax.fori_loop(0, _G // 16, grp, 0)
        cp3.wait()

        def edge(e, _):
            for h in range(_HPC):
                spl = plsc.load_gather(alphabuf, [lane * 0 + h, lane * 0 + e])
                vr[e, pl.ds(32 * h, 16)] = vr[e, pl.ds(32 * h, 16)] * spl
                vr[e, pl.ds(32 * h + 16, 16)] = (
                    vr[e, pl.ds(32 * h + 16, 16)] * spl)
            return 0
        lax.fori_loop(0, _G, edge, 0)

        for h in range(_HPC):
            pltpu.sync_copy(alphabuf.at[h],
                            exr.at[pl.ds(cE4 + h * E + e0 + base, _G)])
        pltpu.sync_copy(exd, den_sh.at[didx], add=True)
        pltpu.sync_copy(vr, acc.at[didx2], add=True)

        def gz(g, _):
            dv = dstall[pl.ds(base + g * 16, 16)]
            col0 = (dv & 7) << 4
            for h in range(_HPC):
                plsc.store_scatter(exd, [g * 16 + lane, col0 + h], zero)
            return 0
        lax.fori_loop(0, _G // 16, gz, 0)
        return 0
    lax.fori_loop(0, _NCH, sweep0, 0)
    plsc.subcore_barrier()

    pltpu.sync_copy(den_sh.at[pl.ds(s * (_DROWS // _TILES), _G)], exd)
    pltpu.sync_copy(exd, den_out.at[c, pl.ds(s * (_DROWS // _TILES), _G)])
    for t in range(_HROWS_PT // _G):
        pltpu.sync_copy(acc.at[pl.ds(s * _HROWS_PT + t * _G, _G)], vr)
        pltpu.sync_copy(vr, msg_un.at[c, pl.ds(s * _HROWS_PT + t * _G, _G)])
    plsc.subcore_barrier()

    zbuf(ks)
    for t in range(_HROWS_PT // _G):
        pltpu.sync_copy(ks, acc.at[pl.ds(s * _HROWS_PT + t * _G, _G)])

    @pl.when(s == _TILES - 1)
    def _zero_dump2():
        pltpu.sync_copy(ks.at[pl.ds(0, 8)], acc.at[pl.ds(_NHALF, 8)])
    plsc.subcore_barrier()

    def sweep1(ci, _):
        base = ci * _G
        copy_idx(srcall, base, vidx, off=4 * N + cN)
        copy_idx(dstall, base, didx2, half=1)
        cpv = pltpu.async_copy(table.at[vidx], vr, sem1)
        for h in range(_HPC):
            pltpu.sync_copy(exr.at[pl.ds(cE4 + h * E + e0 + base, _G)],
                            alphabuf.at[h])
        cpv.wait()

        def edge(e, _):
            for h in range(_HPC):
                spl = plsc.load_gather(alphabuf, [lane * 0 + h, lane * 0 + e])
                vr[e, pl.ds(32 * h, 16)] = vr[e, pl.ds(32 * h, 16)] * spl
                vr[e, pl.ds(32 * h + 16, 16)] = (
                    vr[e, pl.ds(32 * h + 16, 16)] * spl)
            return 0
        lax.fori_loop(0, _G, edge, 0)
        pltpu.sync_copy(vr, acc.at[didx2], add=True)
        return 0
    lax.fori_loop(0, _NCH, sweep1, 0)
    plsc.subcore_barrier()

    for t in range(_HROWS_PT // _G):
        pltpu.sync_copy(acc.at[pl.ds(s * _HROWS_PT + t * _G, _G)], vr)
        pltpu.sync_copy(
            vr, msg_un.at[c, pl.ds(_NHALF + s * _HROWS_PT + t * _G, _G)])


_edge_sc = pl.kernel(
    _edge_body,
    out_type=[
        jax.ShapeDtypeStruct((2, _NPAD, _HC), jnp.float32),
        jax.ShapeDtypeStruct((2, _DROWS, _HC), jnp.float32),
        jax.ShapeDtypeStruct((2 * 4 * E,), jnp.float32),
    ],
    mesh=plsc.VectorSubcoreMesh(core_axis_name="c", subcore_axis_name="s"),
    compiler_params=pltpu.CompilerParams(needs_layout_passes=False),
    scratch_types=[
        pltpu.VMEM((_EPT,), jnp.int32),
        pltpu.VMEM((_EPT,), jnp.int32),
        pltpu.VMEM((_G, _HC), jnp.float32),
        pltpu.VMEM((_G, _HC), jnp.float32),
        pltpu.VMEM((_G, _HC), jnp.float32),
        pltpu.VMEM((_G, _HC), jnp.float32),
        pltpu.VMEM((_HPC, _G), jnp.float32),
        pltpu.VMEM((_G,), jnp.int32),
        pltpu.VMEM((_G,), jnp.int32),
        pltpu.VMEM((_G,), jnp.int32),
        pltpu.VMEM((_G,), jnp.int32),
        pltpu.VMEM((_G,), jnp.int32),
        pltpu.VMEM_SHARED((_NHALF + 8, _HC), jnp.float32),
        pltpu.VMEM_SHARED((_DROWS, _HC), jnp.float32),
        pltpu.SemaphoreType.DMA,
        pltpu.SemaphoreType.DMA,
        pltpu.SemaphoreType.DMA,
    ],
)


def _edge_phase(table, src, dst):
    msg_un, den, _ = _edge_sc(table, src, dst)
    msg = msg_un.transpose(1, 0, 2).reshape(_NPAD, C)[:N]
    d = den.reshape(2, _DROWS, 8, 16)[:, :, :, :_HPC]
    d = d.reshape(2, _NPAD, _HPC)[:, :N]
    den8 = jnp.concatenate([d[0], d[1]], axis=1)
    return msg, den8


def kernel(x, edge_index, params):
    src = edge_index[0].astype(jnp.int32)
    dst = edge_index[1].astype(jnp.int32)
    x = _mm_bias(x, params['in_W'], params['in_b'])
    for l in range(2):
        p = params['blocks'][l]
        wqkvs = jnp.concatenate([p['Wq'], p['Wk'], p['Wv'], p['Wskip']], axis=1)
        bqkvs = jnp.concatenate([p['bq'], p['bk'], p['bv'], p['bskip']])
        qkvs = _mm_bias(x, wqkvs, bqkvs)
        skip = qkvs[:, 3 * C:]
        table = (qkvs[:, :3 * C].reshape(N, 6, _HC)
                 .transpose(1, 0, 2).reshape(6 * N, _HC))
        msg, den8 = _edge_phase(table, src, dst)
        pp = dict(p)
        pp['bf2'] = p['b2']
        x = _post_block(x, msg, den8, skip, pp)
    return _mm_bias(x, params['out_W'], params['out_b'])
